# baseline (speedup 1.0000x reference)
import jax
import jax.numpy as jnp
from jax.experimental import pallas as pl
from jax.experimental.pallas import tpu as pltpu

IN_DIM = 1024
EMBED = 64
NTOK = 32
KCODES = 1024
BATCH = 4096
HID = 512

BLK = 256
GRID = BATCH // BLK

_INV_SQRT2 = 0.7071067811865476


def _gelu(v):
    return 0.5 * v * (1.0 + jax.lax.erf(v * _INV_SQRT2))


def _dot(a, b):
    return jnp.dot(a, b, preferred_element_type=jnp.float32)


def _fused_kernel(x_ref, eW1, eb1, eW2, eb2, eW3, eb3,
                  dW1, db1, dW2, db2, dW3, db3,
                  cb_ref, cbT_ref,
                  rec_ref, idx_ref, closs_ref,
                  qz_ref):
    bf16 = jnp.bfloat16
    x = x_ref[...]
    h = _gelu(_dot(x, eW1[...]) + eb1[...])
    h = _gelu(_dot(h.astype(bf16), eW2[...]) + eb2[...])
    z = _dot(h.astype(bf16), eW3[...]) + eb3[...]
    z_bf = z.astype(bf16)

    cbT = cbT_ref[...]
    cb_sq = jnp.sum(cbT * cbT, axis=0, keepdims=True)
    cbT2 = cbT.astype(bf16) * bf16(-2.0)
    cb_bf = cb_ref[...].astype(bf16)

    ii = jax.lax.broadcasted_iota(jnp.int32, (BLK, KCODES), 1)
    tt = jax.lax.broadcasted_iota(jnp.int32, (BLK, NTOK), 1)

    idx_mat = jnp.zeros((BLK, NTOK), dtype=jnp.int32)
    closs_acc = jnp.float32(0.0)
    one_bf = jnp.ones((), bf16)
    zero_bf = jnp.zeros((), bf16)
    for t in range(NTOK):
        f = z[:, EMBED * t:EMBED * (t + 1)]
        f_sq = jnp.sum(f * f, axis=1, keepdims=True)
        d = (f_sq + _dot(z_bf[:, EMBED * t:EMBED * (t + 1)], cbT2)) + cb_sq
        idx_t = jnp.argmin(d, axis=1).astype(jnp.int32)
        idx_mat = jnp.where(tt == t, idx_t[:, None], idx_mat)
        onehot = (ii == idx_t[:, None]).astype(jnp.float32).astype(bf16)
        q = _dot(onehot, cb_bf)
        closs_acc += jnp.sum((q - f) ** 2)
        qz_ref[:, EMBED * t:EMBED * (t + 1)] = q.astype(bf16)

    idx_ref[...] = idx_mat

    qz = qz_ref[...]
    r = _gelu(_dot(qz, dW1[...]) + db1[...])
    r = _gelu(_dot(r.astype(bf16), dW2[...]) + db2[...])
    rec_ref[...] = _dot(r.astype(bf16), dW3[...]) + db3[...]

    @pl.when(pl.program_id(0) == 0)
    def _init():
        closs_ref[...] = jnp.zeros_like(closs_ref)

    closs_ref[...] += closs_acc


def kernel(x, enc_W1, enc_b1, enc_W2, enc_b2, enc_W3, enc_b3,
           dec_W1, dec_b1, dec_W2, dec_b2, dec_W3, dec_b3, codebook):
    bf16 = jnp.bfloat16
    cbT = codebook.T
    full = lambda shape: pl.BlockSpec(shape, lambda i: (0, 0))
    row = lambda n: pl.BlockSpec((1, n), lambda i: (0, 0))

    rec, idx, closs = pl.pallas_call(
        _fused_kernel,
        grid=(GRID,),
        in_specs=[
            pl.BlockSpec((BLK, IN_DIM), lambda i: (i, 0)),
            full((IN_DIM, HID)), row(HID),
            full((HID, HID)), row(HID),
            full((HID, EMBED * NTOK)), row(EMBED * NTOK),
            full((EMBED * NTOK, HID)), row(HID),
            full((HID, HID)), row(HID),
            full((HID, IN_DIM)), row(IN_DIM),
            full((KCODES, EMBED)),
            full((EMBED, KCODES)),
        ],
        out_specs=[
            pl.BlockSpec((BLK, IN_DIM), lambda i: (i, 0)),
            pl.BlockSpec((BLK, NTOK), lambda i: (i, 0)),
            pl.BlockSpec((8, 128), lambda i: (0, 0)),
        ],
        out_shape=[
            jax.ShapeDtypeStruct((BATCH, IN_DIM), jnp.float32),
            jax.ShapeDtypeStruct((BATCH, NTOK), jnp.int32),
            jax.ShapeDtypeStruct((8, 128), jnp.float32),
        ],
        scratch_shapes=[pltpu.VMEM((BLK, EMBED * NTOK), jnp.bfloat16)],
        compiler_params=pltpu.CompilerParams(
            dimension_semantics=("arbitrary",),
        ),
    )(x.astype(bf16),
      enc_W1.astype(bf16), enc_b1.reshape(1, HID),
      enc_W2.astype(bf16), enc_b2.reshape(1, HID),
      enc_W3.astype(bf16), enc_b3.reshape(1, EMBED * NTOK),
      dec_W1.astype(bf16), dec_b1.reshape(1, HID),
      dec_W2.astype(bf16), dec_b2.reshape(1, HID),
      dec_W3.astype(bf16), dec_b3.reshape(1, IN_DIM),
      codebook, cbT)

    commit_loss = closs[0, 0] / jnp.float32(BATCH * NTOK * EMBED)
    return rec, idx, commit_loss


# speedup vs baseline: 1.4940x; 1.4940x over previous
import jax
import jax.numpy as jnp
from jax.experimental import pallas as pl
from jax.experimental.pallas import tpu as pltpu

IN_DIM = 1024
EMBED = 64
NTOK = 32
KCODES = 1024
BATCH = 4096
HID = 512

BLK = 256
GRID = BATCH // BLK

_INV_SQRT2 = 0.7071067811865476


def _gelu(v):
    return 0.5 * v * (1.0 + jax.lax.erf(v * _INV_SQRT2))


def _dot(a, b):
    return jnp.dot(a, b, preferred_element_type=jnp.float32)


def _fused_kernel(x_ref, eW1, eb1, eW2, eb2, eW3, eb3,
                  dW1, db1, dW2, db2, dW3, db3,
                  cb_ref, cbT_ref,
                  rec_ref, idx_ref, closs_ref,
                  qz_ref):
    x = x_ref[...]
    h = _gelu(_dot(x, eW1[...]) + eb1[...])
    h = _gelu(_dot(h, eW2[...]) + eb2[...])
    z = _dot(h, eW3[...]) + eb3[...]

    cb = cb_ref[...]
    cbT = cbT_ref[...]
    cb_sq = jnp.sum(cbT * cbT, axis=0, keepdims=True)
    cbT2 = cbT * -2.0

    ii = jax.lax.broadcasted_iota(jnp.int32, (BLK, KCODES), 1)
    tt = jax.lax.broadcasted_iota(jnp.int32, (BLK, NTOK), 1)

    idx_mat = jnp.zeros((BLK, NTOK), dtype=jnp.int32)
    closs_acc = jnp.float32(0.0)
    for t in range(NTOK):
        f = z[:, EMBED * t:EMBED * (t + 1)]
        f_sq = jnp.sum(f * f, axis=1, keepdims=True)
        d = (f_sq + _dot(f, cbT2)) + cb_sq
        idx_t = jnp.argmin(d, axis=1).astype(jnp.int32)
        idx_mat = jnp.where(tt == t, idx_t[:, None], idx_mat)
        onehot = (ii == idx_t[:, None]).astype(jnp.float32)
        q = _dot(onehot, cb)
        closs_acc += jnp.sum((q - f) ** 2)
        qz_ref[:, EMBED * t:EMBED * (t + 1)] = q

    idx_ref[...] = idx_mat

    qz = qz_ref[...]
    r = _gelu(_dot(qz, dW1[...]) + db1[...])
    r = _gelu(_dot(r, dW2[...]) + db2[...])
    rec_ref[...] = _dot(r, dW3[...]) + db3[...]

    @pl.when(pl.program_id(0) == 0)
    def _init():
        closs_ref[...] = jnp.zeros_like(closs_ref)

    closs_ref[...] += closs_acc


def kernel(x, enc_W1, enc_b1, enc_W2, enc_b2, enc_W3, enc_b3,
           dec_W1, dec_b1, dec_W2, dec_b2, dec_W3, dec_b3, codebook):
    cbT = codebook.T
    full = lambda shape: pl.BlockSpec(shape, lambda i: (0, 0))
    row = lambda n: pl.BlockSpec((1, n), lambda i: (0, 0))

    rec, idx, closs = pl.pallas_call(
        _fused_kernel,
        grid=(GRID,),
        in_specs=[
            pl.BlockSpec((BLK, IN_DIM), lambda i: (i, 0)),
            full((IN_DIM, HID)), row(HID),
            full((HID, HID)), row(HID),
            full((HID, EMBED * NTOK)), row(EMBED * NTOK),
            full((EMBED * NTOK, HID)), row(HID),
            full((HID, HID)), row(HID),
            full((HID, IN_DIM)), row(IN_DIM),
            full((KCODES, EMBED)),
            full((EMBED, KCODES)),
        ],
        out_specs=[
            pl.BlockSpec((BLK, IN_DIM), lambda i: (i, 0)),
            pl.BlockSpec((BLK, NTOK), lambda i: (i, 0)),
            pl.BlockSpec((8, 128), lambda i: (0, 0)),
        ],
        out_shape=[
            jax.ShapeDtypeStruct((BATCH, IN_DIM), jnp.float32),
            jax.ShapeDtypeStruct((BATCH, NTOK), jnp.int32),
            jax.ShapeDtypeStruct((8, 128), jnp.float32),
        ],
        scratch_shapes=[pltpu.VMEM((BLK, EMBED * NTOK), jnp.float32)],
        compiler_params=pltpu.CompilerParams(
            dimension_semantics=("arbitrary",),
        ),
    )(x,
      enc_W1, enc_b1.reshape(1, HID),
      enc_W2, enc_b2.reshape(1, HID),
      enc_W3, enc_b3.reshape(1, EMBED * NTOK),
      dec_W1, dec_b1.reshape(1, HID),
      dec_W2, dec_b2.reshape(1, HID),
      dec_W3, dec_b3.reshape(1, IN_DIM),
      codebook, cbT)

    commit_loss = closs[0, 0] / jnp.float32(BATCH * NTOK * EMBED)
    return rec, idx, commit_loss


# speedup vs baseline: 1.5726x; 1.0526x over previous
import jax
import jax.numpy as jnp
from jax.experimental import pallas as pl
from jax.experimental.pallas import tpu as pltpu

IN_DIM = 1024
EMBED = 64
NTOK = 32
KCODES = 1024
BATCH = 4096
HID = 512

BLK = 512
GRID = BATCH // BLK

_INV_SQRT2 = 0.7071067811865476


def _gelu(v):
    return 0.5 * v * (1.0 + jax.lax.erf(v * _INV_SQRT2))


def _dot(a, b):
    return jnp.dot(a, b, preferred_element_type=jnp.float32)


def _fused_kernel(x_ref, eW1, eb1, eW2, eb2, eW3, eb3,
                  dW1, db1, dW2, db2, dW3, db3,
                  cb_ref, cbT_ref,
                  rec_ref, idx_ref, closs_ref,
                  qz_ref):
    x = x_ref[...]
    h = _gelu(_dot(x, eW1[...]) + eb1[...])
    h = _gelu(_dot(h, eW2[...]) + eb2[...])
    z = _dot(h, eW3[...]) + eb3[...]

    cb = cb_ref[...]
    cbT = cbT_ref[...]
    cb_sq = jnp.sum(cbT * cbT, axis=0, keepdims=True)
    cbT2 = cbT * -2.0

    ii = jax.lax.broadcasted_iota(jnp.int32, (BLK, KCODES), 1)
    tt = jax.lax.broadcasted_iota(jnp.int32, (BLK, NTOK), 1)

    idx_mat = jnp.zeros((BLK, NTOK), dtype=jnp.int32)
    closs_acc = jnp.float32(0.0)
    for t in range(NTOK):
        f = z[:, EMBED * t:EMBED * (t + 1)]
        f_sq = jnp.sum(f * f, axis=1, keepdims=True)
        d = (f_sq + _dot(f, cbT2)) + cb_sq
        idx_t = jnp.argmin(d, axis=1).astype(jnp.int32)
        idx_mat = jnp.where(tt == t, idx_t[:, None], idx_mat)
        onehot = (ii == idx_t[:, None]).astype(jnp.float32)
        q = _dot(onehot, cb)
        closs_acc += jnp.sum((q - f) ** 2)
        qz_ref[:, EMBED * t:EMBED * (t + 1)] = q

    idx_ref[...] = idx_mat

    qz = qz_ref[...]
    r = _gelu(_dot(qz, dW1[...]) + db1[...])
    r = _gelu(_dot(r, dW2[...]) + db2[...])
    rec_ref[...] = _dot(r, dW3[...]) + db3[...]

    @pl.when(pl.program_id(0) == 0)
    def _init():
        closs_ref[...] = jnp.zeros_like(closs_ref)

    closs_ref[...] += closs_acc


def kernel(x, enc_W1, enc_b1, enc_W2, enc_b2, enc_W3, enc_b3,
           dec_W1, dec_b1, dec_W2, dec_b2, dec_W3, dec_b3, codebook):
    cbT = codebook.T
    full = lambda shape: pl.BlockSpec(shape, lambda i: (0, 0))
    row = lambda n: pl.BlockSpec((1, n), lambda i: (0, 0))

    rec, idx, closs = pl.pallas_call(
        _fused_kernel,
        grid=(GRID,),
        in_specs=[
            pl.BlockSpec((BLK, IN_DIM), lambda i: (i, 0)),
            full((IN_DIM, HID)), row(HID),
            full((HID, HID)), row(HID),
            full((HID, EMBED * NTOK)), row(EMBED * NTOK),
            full((EMBED * NTOK, HID)), row(HID),
            full((HID, HID)), row(HID),
            full((HID, IN_DIM)), row(IN_DIM),
            full((KCODES, EMBED)),
            full((EMBED, KCODES)),
        ],
        out_specs=[
            pl.BlockSpec((BLK, IN_DIM), lambda i: (i, 0)),
            pl.BlockSpec((BLK, NTOK), lambda i: (i, 0)),
            pl.BlockSpec((8, 128), lambda i: (0, 0)),
        ],
        out_shape=[
            jax.ShapeDtypeStruct((BATCH, IN_DIM), jnp.float32),
            jax.ShapeDtypeStruct((BATCH, NTOK), jnp.int32),
            jax.ShapeDtypeStruct((8, 128), jnp.float32),
        ],
        scratch_shapes=[pltpu.VMEM((BLK, EMBED * NTOK), jnp.float32)],
        compiler_params=pltpu.CompilerParams(
            dimension_semantics=("arbitrary",),
        ),
    )(x,
      enc_W1, enc_b1.reshape(1, HID),
      enc_W2, enc_b2.reshape(1, HID),
      enc_W3, enc_b3.reshape(1, EMBED * NTOK),
      dec_W1, dec_b1.reshape(1, HID),
      dec_W2, dec_b2.reshape(1, HID),
      dec_W3, dec_b3.reshape(1, IN_DIM),
      codebook, cbT)

    commit_loss = closs[0, 0] / jnp.float32(BATCH * NTOK * EMBED)
    return rec, idx, commit_loss


# speedup vs baseline: 1.5754x; 1.0018x over previous
import jax
import jax.numpy as jnp
from jax.experimental import pallas as pl
from jax.experimental.pallas import tpu as pltpu

IN_DIM = 1024
EMBED = 64
NTOK = 32
KCODES = 1024
BATCH = 4096
HID = 512

BLK = 512
GRID = BATCH // BLK

_INV_SQRT2 = 0.7071067811865476


def _gelu(v):
    return 0.5 * v * (1.0 + jax.lax.erf(v * _INV_SQRT2))


def _dot(a, b):
    return jnp.dot(a, b, preferred_element_type=jnp.float32)


def _fused_kernel(x_ref, eW1, eb1, eW2, eb2, eW3, eb3,
                  dW1, db1, dW2, db2, dW3, db3,
                  cb_ref, cbT_ref,
                  rec_ref, idx_ref, closs_ref,
                  qz_ref):
    x = x_ref[...]
    h = _gelu(_dot(x, eW1[...]) + eb1[...])
    h = _gelu(_dot(h, eW2[...]) + eb2[...])
    z = _dot(h, eW3[...]) + eb3[...]

    cb = cb_ref[...]
    cbT = cbT_ref[...]
    cb_sq = jnp.sum(cbT * cbT, axis=0, keepdims=True)
    cbT2 = cbT * -2.0

    ii = jax.lax.broadcasted_iota(jnp.int32, (BLK, KCODES), 1)
    tt = jax.lax.broadcasted_iota(jnp.int32, (BLK, NTOK), 1)

    idx_mat = jnp.zeros((BLK, NTOK), dtype=jnp.int32)
    closs_acc = jnp.float32(0.0)
    for t in range(NTOK):
        f = z[:, EMBED * t:EMBED * (t + 1)]
        f_sq = jnp.sum(f * f, axis=1, keepdims=True)
        d = (f_sq + _dot(f, cbT2)) + cb_sq
        idx_t = jnp.argmin(d, axis=1).astype(jnp.int32)
        idx_mat = jnp.where(tt == t, idx_t[:, None], idx_mat)
        onehot = (ii == idx_t[:, None]).astype(jnp.float32)
        q = _dot(onehot, cb)
        closs_acc += jnp.sum((q - f) ** 2)
        qz_ref[:, EMBED * t:EMBED * (t + 1)] = q

    idx_ref[...] = idx_mat

    qz = qz_ref[...]
    r = _gelu(_dot(qz, dW1[...]) + db1[...])
    r = _gelu(_dot(r, dW2[...]) + db2[...])
    rec_ref[...] = _dot(r, dW3[...]) + db3[...]

    closs_ref[...] = jnp.broadcast_to(closs_acc, (1, 1, 128))


def kernel(x, enc_W1, enc_b1, enc_W2, enc_b2, enc_W3, enc_b3,
           dec_W1, dec_b1, dec_W2, dec_b2, dec_W3, dec_b3, codebook):
    cbT = codebook.T
    full = lambda shape: pl.BlockSpec(shape, lambda i: (0, 0))
    row = lambda n: pl.BlockSpec((1, n), lambda i: (0, 0))

    rec, idx, closs = pl.pallas_call(
        _fused_kernel,
        grid=(GRID,),
        in_specs=[
            pl.BlockSpec((BLK, IN_DIM), lambda i: (i, 0)),
            full((IN_DIM, HID)), row(HID),
            full((HID, HID)), row(HID),
            full((HID, EMBED * NTOK)), row(EMBED * NTOK),
            full((EMBED * NTOK, HID)), row(HID),
            full((HID, HID)), row(HID),
            full((HID, IN_DIM)), row(IN_DIM),
            full((KCODES, EMBED)),
            full((EMBED, KCODES)),
        ],
        out_specs=[
            pl.BlockSpec((BLK, IN_DIM), lambda i: (i, 0)),
            pl.BlockSpec((BLK, NTOK), lambda i: (i, 0)),
            pl.BlockSpec((1, 1, 128), lambda i: (i, 0, 0)),
        ],
        out_shape=[
            jax.ShapeDtypeStruct((BATCH, IN_DIM), jnp.float32),
            jax.ShapeDtypeStruct((BATCH, NTOK), jnp.int32),
            jax.ShapeDtypeStruct((GRID, 1, 128), jnp.float32),
        ],
        scratch_shapes=[pltpu.VMEM((BLK, EMBED * NTOK), jnp.float32)],
        compiler_params=pltpu.CompilerParams(
            dimension_semantics=("parallel",),
        ),
    )(x,
      enc_W1, enc_b1.reshape(1, HID),
      enc_W2, enc_b2.reshape(1, HID),
      enc_W3, enc_b3.reshape(1, EMBED * NTOK),
      dec_W1, dec_b1.reshape(1, HID),
      dec_W2, dec_b2.reshape(1, HID),
      dec_W3, dec_b3.reshape(1, IN_DIM),
      codebook, cbT)

    commit_loss = jnp.sum(closs[:, 0, 0]) / jnp.float32(BATCH * NTOK * EMBED)
    return rec, idx, commit_loss
